# baseline (speedup 1.0000x reference)
import functools

import jax
import jax.numpy as jnp
from jax import lax
from jax.experimental import pallas as pl
from jax.experimental.pallas import tpu as pltpu
from jax.experimental.pallas import tpu_sc as plsc

N_NODES = 10000
N_EDGES = 160000
D_IN, D_H, D_OUT = 256, 512, 256

NUM_CORES = 2
NUM_TILES = 16
LANES = 16
F = 64

E_PER_TILE = N_EDGES // NUM_TILES
EDGE_BATCH = 80
N_BATCHES = E_PER_TILE // EDGE_BATCH
WB = 80
N_WBLOCKS = N_NODES // WB
WBLOCKS_PER_TILE = -(-N_WBLOCKS // NUM_TILES)


NBUF = 5

_GATHER_DNUMS = lax.GatherDimensionNumbers(
    offset_dims=(), collapsed_slice_dims=(0,), start_index_map=(0,)
)


def _lane_splat(vv, k):
    idx = jnp.full((LANES, 1), k, jnp.int32)
    return lax.gather(
        vv, idx, _GATHER_DNUMS, (1,),
        mode=lax.GatherScatterMode.PROMISE_IN_BOUNDS,
    )


def _make_segsum(n_chunks):
    mesh = plsc.VectorSubcoreMesh(core_axis_name="c", subcore_axis_name="s")

    @functools.partial(
        pl.kernel,
        mesh=mesh,
        compiler_params=pltpu.CompilerParams(use_tc_tiling_on_sc=False),
        out_type=jax.ShapeDtypeStruct((N_NODES, n_chunks * F), jnp.float32),
        scratch_types=[
            pltpu.VMEM((N_BATCHES, EDGE_BATCH), jnp.int32),
            pltpu.VMEM((N_BATCHES, EDGE_BATCH), jnp.int32),
            pltpu.VMEM((N_BATCHES, EDGE_BATCH), jnp.float32),
            [pltpu.VMEM((EDGE_BATCH, F), jnp.float32)] * NBUF,
            pltpu.VMEM((WB, F), jnp.float32),
            pltpu.VMEM_SHARED((N_NODES, F), jnp.float32),
            [pltpu.SemaphoreType.DMA] * NBUF,
            [pltpu.SemaphoreType.DMA] * NBUF,
            pltpu.SemaphoreType.DMA,
        ],
    )
    def seg(*refs):
        h_hbm, col_hbm, row_hbm, vals_hbm, out_hbm = refs[:5]
        (col_v, row_v, vals_v, rows, stage_v, agg_s,
         gsem, ssem, wsem) = refs[5:]

        cid = lax.axis_index("c")
        sid = lax.axis_index("s")

        pltpu.sync_copy(col_hbm.at[sid], col_v)
        pltpu.sync_copy(row_hbm.at[sid], row_v)
        pltpu.sync_copy(vals_hbm.at[sid], vals_v)

        zero = jnp.zeros((LANES,), jnp.float32)

        def zrow(i, carry):
            for q in range(F // LANES):
                stage_v[i, pl.ds(q * LANES, LANES)] = zero
            return carry

        lax.fori_loop(0, WB, zrow, 0)

        def start_gather(h_hbm, b, j):
            pltpu.async_copy(h_hbm.at[col_v.at[j]], rows[b], gsem[b])

        def wait_gather(h_hbm, b, j):
            pltpu.make_async_copy(h_hbm.at[col_v.at[j]], rows[b], gsem[b]).wait()

        def start_scatter(b, j):
            pltpu.async_copy(rows[b], agg_s.at[row_v.at[j]], ssem[b], add=True)

        def wait_scatter(b, j):
            pltpu.make_async_copy(rows[b], agg_s.at[row_v.at[j]], ssem[b]).wait()

        def scale(b, j):
            @plsc.parallel_loop(0, EDGE_BATCH // LANES, unroll=5)
            def body(g):
                vv = vals_v[j, pl.ds(g * LANES, LANES)]
                for k in range(LANES):
                    v = _lane_splat(vv, k)
                    e = g * LANES + k
                    for q in range(F // LANES):
                        sl = pl.ds(q * LANES, LANES)
                        rows[b][e, sl] = rows[b][e, sl] * v

        def chunk_pass(kk, carry):
            pc = NUM_CORES * kk + cid
            h_p = h_hbm.at[pc]

            for k in range(WBLOCKS_PER_TILE):
                t = sid + NUM_TILES * k

                @pl.when(t < N_WBLOCKS)
                def _(t=t):
                    pltpu.async_copy(stage_v, agg_s.at[pl.ds(t * WB, WB)], wsem)

            for k in range(WBLOCKS_PER_TILE):
                t = sid + NUM_TILES * k

                @pl.when(t < N_WBLOCKS)
                def _(t=t):
                    pltpu.make_async_copy(
                        stage_v, agg_s.at[pl.ds(t * WB, WB)], wsem
                    ).wait()

            plsc.subcore_barrier()

            for b in range(3):
                start_gather(h_p, b, b)

            for i in range(NBUF):
                wait_gather(h_p, i, i)
                scale(i, i)
                start_scatter(i, i)
                bp = (i + 3) % NBUF
                if i >= 2:
                    wait_scatter(bp, bp)
                start_gather(h_p, bp, i + 3)

            def round_(t, carry2):
                for i in range(NBUF):
                    j = t * NBUF + i
                    wait_gather(h_p, i, j)
                    scale(i, j)
                    start_scatter(i, j)

                    @pl.when(j + 3 < N_BATCHES)
                    def _(i=i, j=j):
                        bp = (i + 3) % NBUF
                        wait_scatter(bp, j - 2)
                        start_gather(h_p, bp, j + 3)

                return carry2

            lax.fori_loop(1, N_BATCHES // NBUF, round_, 0)

            for b in range(NBUF):
                wait_scatter(b, N_BATCHES - NBUF + b)

            plsc.subcore_barrier()

            for k in range(WBLOCKS_PER_TILE):
                t = sid + NUM_TILES * k

                @pl.when(t < N_WBLOCKS)
                def _(t=t):
                    pltpu.async_copy(
                        agg_s.at[pl.ds(t * WB, WB)],
                        out_hbm.at[pl.ds(t * WB, WB), pl.ds(pc * F, F)],
                        wsem,
                    )

            for k in range(WBLOCKS_PER_TILE):
                t = sid + NUM_TILES * k

                @pl.when(t < N_WBLOCKS)
                def _(t=t):
                    pltpu.make_async_copy(
                        agg_s.at[pl.ds(t * WB, WB)],
                        out_hbm.at[pl.ds(t * WB, WB), pl.ds(pc * F, F)],
                        wsem,
                    ).wait()

            plsc.subcore_barrier()
            return carry

        lax.fori_loop(0, n_chunks // NUM_CORES, chunk_pass, 0)

    return seg


_segsum_l1 = _make_segsum(D_H // F)
_segsum_l2 = _make_segsum(D_OUT // F)

_RB = 2000


def _make_mm(d_in, d_out, relu):
    n_chunks = d_out // F

    def body(x_ref, w_ref, o_ref):
        xb = x_ref[...]
        if relu:
            xb = jnp.maximum(xb, 0.0)
        d = lax.dot_general(
            xb, w_ref[...],
            (((1,), (1,)), ((), ())),
            preferred_element_type=jnp.float32,
        )
        for c in range(n_chunks):
            o_ref[c] = d[:, c * F:(c + 1) * F]

    return pl.pallas_call(
        body,
        grid=(N_NODES // _RB,),
        in_specs=[
            pl.BlockSpec((_RB, d_in), lambda i: (i, 0)),
            pl.BlockSpec((d_out, d_in), lambda i: (0, 0)),
        ],
        out_specs=pl.BlockSpec((n_chunks, _RB, F), lambda i: (0, i, 0)),
        out_shape=jax.ShapeDtypeStruct((n_chunks, N_NODES, F), jnp.float32),
    )


def _mm1(x, w):
    return _make_mm(D_IN, D_H, relu=False)(x, w)


def _mm2(a, w):
    return _make_mm(D_H, D_OUT, relu=True)(a, w)


def kernel(x, edge_index, adj_vals, W1, W2):
    col = edge_index[1].reshape(NUM_TILES, N_BATCHES, EDGE_BATCH)
    row = edge_index[0].reshape(NUM_TILES, N_BATCHES, EDGE_BATCH)
    vals = adj_vals.reshape(NUM_TILES, N_BATCHES, EDGE_BATCH)

    h1 = _mm1(x, W1)
    a1 = _segsum_l1(h1, col, row, vals)
    h2 = _mm2(a1, W2)
    return _segsum_l2(h2, col, row, vals)


# speedup vs baseline: 1.1956x; 1.1956x over previous
import functools

import jax
import jax.numpy as jnp
from jax import lax
from jax.experimental import pallas as pl
from jax.experimental.pallas import tpu as pltpu
from jax.experimental.pallas import tpu_sc as plsc

N_NODES = 10000
N_EDGES = 160000
D_IN, D_H, D_OUT = 256, 512, 256

NUM_CORES = 2
NUM_TILES = 16
LANES = 16
F = 64

E_PER_TILE = N_EDGES // NUM_TILES
EDGE_BATCH = 80
N_BATCHES = E_PER_TILE // EDGE_BATCH
WB = 80
N_WBLOCKS = N_NODES // WB
WBLOCKS_PER_TILE = -(-N_WBLOCKS // NUM_TILES)


NBUF = 5

_GATHER_DNUMS = lax.GatherDimensionNumbers(
    offset_dims=(), collapsed_slice_dims=(0,), start_index_map=(0,)
)


def _lane_splat(vv, k):
    idx = jnp.full((LANES, 1), k, jnp.int32)
    return lax.gather(
        vv, idx, _GATHER_DNUMS, (1,),
        mode=lax.GatherScatterMode.PROMISE_IN_BOUNDS,
    )


def _make_segsum(n_chunks):
    mesh = plsc.VectorSubcoreMesh(core_axis_name="c", subcore_axis_name="s")

    @functools.partial(
        pl.kernel,
        mesh=mesh,
        compiler_params=pltpu.CompilerParams(use_tc_tiling_on_sc=False),
        out_type=jax.ShapeDtypeStruct((N_NODES, n_chunks * F), jnp.float32),
        scratch_types=[
            pltpu.VMEM((N_BATCHES, EDGE_BATCH), jnp.int32),
            pltpu.VMEM((N_BATCHES, EDGE_BATCH), jnp.int32),
            pltpu.VMEM((N_BATCHES, EDGE_BATCH), jnp.float32),
            [pltpu.VMEM((EDGE_BATCH, F), jnp.float32)] * NBUF,
            pltpu.VMEM((WB, F), jnp.float32),
            pltpu.VMEM_SHARED((N_NODES, F), jnp.float32),
            [pltpu.SemaphoreType.DMA] * NBUF,
            [pltpu.SemaphoreType.DMA] * NBUF,
            pltpu.SemaphoreType.DMA,
        ],
    )
    def seg(*refs):
        h_hbm, col_hbm, row_hbm, vals_hbm, out_hbm = refs[:5]
        (col_v, row_v, vals_v, rows, stage_v, agg_s,
         gsem, ssem, wsem) = refs[5:]

        cid = lax.axis_index("c")
        sid = lax.axis_index("s")

        pltpu.sync_copy(col_hbm.at[sid], col_v)
        pltpu.sync_copy(row_hbm.at[sid], row_v)
        pltpu.sync_copy(vals_hbm.at[sid], vals_v)

        zero = jnp.zeros((LANES,), jnp.float32)

        def zrow(i, carry):
            for q in range(F // LANES):
                stage_v[i, pl.ds(q * LANES, LANES)] = zero
            return carry

        lax.fori_loop(0, WB, zrow, 0)

        def start_gather(h_hbm, b, j):
            pltpu.async_copy(h_hbm.at[col_v.at[j]], rows[b], gsem[b])

        def wait_gather(h_hbm, b, j):
            pltpu.make_async_copy(h_hbm.at[col_v.at[j]], rows[b], gsem[b]).wait()

        def start_scatter(b, j):
            pltpu.async_copy(rows[b], agg_s.at[row_v.at[j]], ssem[b], add=True)

        def wait_scatter(b, j):
            pltpu.make_async_copy(rows[b], agg_s.at[row_v.at[j]], ssem[b]).wait()

        def scale(b, j):
            @plsc.parallel_loop(0, EDGE_BATCH // LANES, unroll=1)
            def body(g):
                vv = vals_v[j, pl.ds(g * LANES, LANES)]
                for k in range(LANES):
                    v = _lane_splat(vv, k)
                    e = g * LANES + k
                    for q in range(F // LANES):
                        sl = pl.ds(q * LANES, LANES)
                        rows[b][e, sl] = rows[b][e, sl] * v

        def chunk_pass(kk, carry):
            pc = NUM_CORES * kk + cid
            h_p = h_hbm.at[pc]

            for k in range(WBLOCKS_PER_TILE):
                t = sid + NUM_TILES * k

                @pl.when(t < N_WBLOCKS)
                def _(t=t):
                    pltpu.async_copy(stage_v, agg_s.at[pl.ds(t * WB, WB)], wsem)

            for k in range(WBLOCKS_PER_TILE):
                t = sid + NUM_TILES * k

                @pl.when(t < N_WBLOCKS)
                def _(t=t):
                    pltpu.make_async_copy(
                        stage_v, agg_s.at[pl.ds(t * WB, WB)], wsem
                    ).wait()

            plsc.subcore_barrier()

            for b in range(3):
                start_gather(h_p, b, b)

            for i in range(NBUF):
                wait_gather(h_p, i, i)
                scale(i, i)
                start_scatter(i, i)
                bp = (i + 3) % NBUF
                if i >= 2:
                    wait_scatter(bp, bp)
                start_gather(h_p, bp, i + 3)

            def round_(t, carry2):
                for i in range(NBUF):
                    j = t * NBUF + i
                    wait_gather(h_p, i, j)
                    scale(i, j)
                    start_scatter(i, j)

                    @pl.when(j + 3 < N_BATCHES)
                    def _(i=i, j=j):
                        bp = (i + 3) % NBUF
                        wait_scatter(bp, j - 2)
                        start_gather(h_p, bp, j + 3)

                return carry2

            lax.fori_loop(1, N_BATCHES // NBUF, round_, 0)

            for b in range(NBUF):
                wait_scatter(b, N_BATCHES - NBUF + b)

            plsc.subcore_barrier()

            for k in range(WBLOCKS_PER_TILE):
                t = sid + NUM_TILES * k

                @pl.when(t < N_WBLOCKS)
                def _(t=t):
                    pltpu.async_copy(
                        agg_s.at[pl.ds(t * WB, WB)],
                        out_hbm.at[pl.ds(t * WB, WB), pl.ds(pc * F, F)],
                        wsem,
                    )

            for k in range(WBLOCKS_PER_TILE):
                t = sid + NUM_TILES * k

                @pl.when(t < N_WBLOCKS)
                def _(t=t):
                    pltpu.make_async_copy(
                        agg_s.at[pl.ds(t * WB, WB)],
                        out_hbm.at[pl.ds(t * WB, WB), pl.ds(pc * F, F)],
                        wsem,
                    ).wait()

            plsc.subcore_barrier()
            return carry

        lax.fori_loop(0, n_chunks // NUM_CORES, chunk_pass, 0)

    return seg


_segsum_l1 = _make_segsum(D_H // F)
_segsum_l2 = _make_segsum(D_OUT // F)

_RB = 2000


def _make_mm(d_in, d_out, relu):
    n_chunks = d_out // F

    def body(x_ref, w_ref, o_ref):
        xb = x_ref[...]
        if relu:
            xb = jnp.maximum(xb, 0.0)
        d = lax.dot_general(
            xb, w_ref[...],
            (((1,), (1,)), ((), ())),
            preferred_element_type=jnp.float32,
        )
        for c in range(n_chunks):
            o_ref[c] = d[:, c * F:(c + 1) * F]

    return pl.pallas_call(
        body,
        grid=(N_NODES // _RB,),
        in_specs=[
            pl.BlockSpec((_RB, d_in), lambda i: (i, 0)),
            pl.BlockSpec((d_out, d_in), lambda i: (0, 0)),
        ],
        out_specs=pl.BlockSpec((n_chunks, _RB, F), lambda i: (0, i, 0)),
        out_shape=jax.ShapeDtypeStruct((n_chunks, N_NODES, F), jnp.float32),
    )


def _mm1(x, w):
    return _make_mm(D_IN, D_H, relu=False)(x, w)


def _mm2(a, w):
    return _make_mm(D_H, D_OUT, relu=True)(a, w)


def kernel(x, edge_index, adj_vals, W1, W2):
    col = edge_index[1].reshape(NUM_TILES, N_BATCHES, EDGE_BATCH)
    row = edge_index[0].reshape(NUM_TILES, N_BATCHES, EDGE_BATCH)
    vals = adj_vals.reshape(NUM_TILES, N_BATCHES, EDGE_BATCH)

    h1 = _mm1(x, W1)
    a1 = _segsum_l1(h1, col, row, vals)
    h2 = _mm2(a1, W2)
    return _segsum_l2(h2, col, row, vals)


# speedup vs baseline: 1.2192x; 1.0197x over previous
import functools

import jax
import jax.numpy as jnp
from jax import lax
from jax.experimental import pallas as pl
from jax.experimental.pallas import tpu as pltpu
from jax.experimental.pallas import tpu_sc as plsc

N_NODES = 10000
N_EDGES = 160000
D_IN, D_H, D_OUT = 256, 512, 256

NUM_CORES = 2
NUM_TILES = 16
LANES = 16
F = 64

E_PER_TILE = N_EDGES // NUM_TILES
EDGE_BATCH = 80
N_BATCHES = E_PER_TILE // EDGE_BATCH
WB = 80
N_WBLOCKS = N_NODES // WB
WBLOCKS_PER_TILE = -(-N_WBLOCKS // NUM_TILES)


NBUF = 5

_GATHER_DNUMS = lax.GatherDimensionNumbers(
    offset_dims=(), collapsed_slice_dims=(0,), start_index_map=(0,)
)


def _lane_splat(vv, k):
    idx = jnp.full((LANES, 1), k, jnp.int32)
    return lax.gather(
        vv, idx, _GATHER_DNUMS, (1,),
        mode=lax.GatherScatterMode.PROMISE_IN_BOUNDS,
    )


def _make_segsum(n_chunks):
    mesh = plsc.VectorSubcoreMesh(core_axis_name="c", subcore_axis_name="s")

    @functools.partial(
        pl.kernel,
        mesh=mesh,
        compiler_params=pltpu.CompilerParams(use_tc_tiling_on_sc=False),
        out_type=jax.ShapeDtypeStruct((N_NODES, n_chunks * F), jnp.float32),
        scratch_types=[
            pltpu.VMEM((N_BATCHES, EDGE_BATCH), jnp.int32),
            pltpu.VMEM((N_BATCHES, EDGE_BATCH), jnp.int32),
            pltpu.VMEM((N_BATCHES, EDGE_BATCH), jnp.float32),
            [pltpu.VMEM((EDGE_BATCH, F), jnp.float32)] * NBUF,
            pltpu.VMEM((WB, F), jnp.float32),
            pltpu.VMEM_SHARED((N_NODES, F), jnp.float32),
            [pltpu.SemaphoreType.DMA] * NBUF,
            [pltpu.SemaphoreType.DMA] * NBUF,
            pltpu.SemaphoreType.DMA,
        ],
    )
    def seg(*refs):
        h_hbm, col_hbm, row_hbm, vals_hbm, out_hbm = refs[:5]
        (col_v, row_v, vals_v, rows, stage_v, agg_s,
         gsem, ssem, wsem) = refs[5:]

        cid = lax.axis_index("c")
        sid = lax.axis_index("s")

        pltpu.sync_copy(col_hbm.at[sid], col_v)
        pltpu.sync_copy(row_hbm.at[sid], row_v)
        pltpu.sync_copy(vals_hbm.at[sid], vals_v)

        zero = jnp.zeros((LANES,), jnp.float32)

        def zrow(i, carry):
            for q in range(F // LANES):
                stage_v[i, pl.ds(q * LANES, LANES)] = zero
            return carry

        lax.fori_loop(0, WB, zrow, 0)

        def start_gather(h_hbm, b, j):
            pltpu.async_copy(h_hbm.at[col_v.at[j]], rows[b], gsem[b])

        def wait_gather(h_hbm, b, j):
            pltpu.make_async_copy(h_hbm.at[col_v.at[j]], rows[b], gsem[b]).wait()

        def start_scatter(b, j):
            pltpu.async_copy(rows[b], agg_s.at[row_v.at[j]], ssem[b], add=True)

        def wait_scatter(b, j):
            pltpu.make_async_copy(rows[b], agg_s.at[row_v.at[j]], ssem[b]).wait()

        def scale(b, j):
            @plsc.parallel_loop(0, EDGE_BATCH // LANES, unroll=1)
            def body(g):
                vv = vals_v[j, pl.ds(g * LANES, LANES)]
                for k in range(LANES):
                    v = _lane_splat(vv, k)
                    e = g * LANES + k
                    for q in range(F // LANES):
                        sl = pl.ds(q * LANES, LANES)
                        rows[b][e, sl] = rows[b][e, sl] * v

        n_passes = n_chunks // NUM_CORES

        def chunk_pass(kk, carry):
            pc = NUM_CORES * kk + cid
            h_p = h_hbm.at[pc]

            for k in range(WBLOCKS_PER_TILE):
                t = sid + NUM_TILES * k

                @pl.when(t < N_WBLOCKS)
                def _(t=t):
                    pltpu.async_copy(stage_v, agg_s.at[pl.ds(t * WB, WB)], wsem)

            for k in range(WBLOCKS_PER_TILE):
                t = sid + NUM_TILES * k

                @pl.when(t < N_WBLOCKS)
                def _(t=t):
                    pltpu.make_async_copy(
                        stage_v, agg_s.at[pl.ds(t * WB, WB)], wsem
                    ).wait()

            plsc.subcore_barrier()

            for i in range(NBUF):
                wait_gather(h_p, i, i)
                scale(i, i)
                start_scatter(i, i)
                bp = (i + 3) % NBUF
                if i >= 2:
                    wait_scatter(bp, bp)
                start_gather(h_p, bp, i + 3)

            def round_(t, carry2):
                for i in range(NBUF):
                    j = t * NBUF + i
                    wait_gather(h_p, i, j)
                    scale(i, j)
                    start_scatter(i, j)

                    @pl.when(j + 3 < N_BATCHES)
                    def _(i=i, j=j):
                        bp = (i + 3) % NBUF
                        wait_scatter(bp, j - 2)
                        start_gather(h_p, bp, j + 3)

                return carry2

            lax.fori_loop(1, N_BATCHES // NBUF, round_, 0)

            for b in range(NBUF):
                wait_scatter(b, N_BATCHES - NBUF + b)

            @pl.when(kk + 1 < n_passes)
            def _():
                h_n = h_hbm.at[pc + NUM_CORES]
                for b in range(3):
                    start_gather(h_n, b, b)

            plsc.subcore_barrier()

            for k in range(WBLOCKS_PER_TILE):
                t = sid + NUM_TILES * k

                @pl.when(t < N_WBLOCKS)
                def _(t=t):
                    pltpu.async_copy(
                        agg_s.at[pl.ds(t * WB, WB)],
                        out_hbm.at[pl.ds(t * WB, WB), pl.ds(pc * F, F)],
                        wsem,
                    )

            for k in range(WBLOCKS_PER_TILE):
                t = sid + NUM_TILES * k

                @pl.when(t < N_WBLOCKS)
                def _(t=t):
                    pltpu.make_async_copy(
                        agg_s.at[pl.ds(t * WB, WB)],
                        out_hbm.at[pl.ds(t * WB, WB), pl.ds(pc * F, F)],
                        wsem,
                    ).wait()

            plsc.subcore_barrier()
            return carry

        for b in range(3):
            start_gather(h_hbm.at[cid], b, b)

        lax.fori_loop(0, n_passes, chunk_pass, 0)

    return seg


_segsum_l1 = _make_segsum(D_H // F)
_segsum_l2 = _make_segsum(D_OUT // F)

_RB = 2000


def _make_mm(d_in, d_out, relu):
    n_chunks = d_out // F

    def body(x_ref, w_ref, o_ref):
        xb = x_ref[...]
        if relu:
            xb = jnp.maximum(xb, 0.0)
        d = lax.dot_general(
            xb, w_ref[...],
            (((1,), (1,)), ((), ())),
            preferred_element_type=jnp.float32,
        )
        for c in range(n_chunks):
            o_ref[c] = d[:, c * F:(c + 1) * F]

    return pl.pallas_call(
        body,
        grid=(N_NODES // _RB,),
        in_specs=[
            pl.BlockSpec((_RB, d_in), lambda i: (i, 0)),
            pl.BlockSpec((d_out, d_in), lambda i: (0, 0)),
        ],
        out_specs=pl.BlockSpec((n_chunks, _RB, F), lambda i: (0, i, 0)),
        out_shape=jax.ShapeDtypeStruct((n_chunks, N_NODES, F), jnp.float32),
    )


def _mm1(x, w):
    return _make_mm(D_IN, D_H, relu=False)(x, w)


def _mm2(a, w):
    return _make_mm(D_H, D_OUT, relu=True)(a, w)


def kernel(x, edge_index, adj_vals, W1, W2):
    col = edge_index[1].reshape(NUM_TILES, N_BATCHES, EDGE_BATCH)
    row = edge_index[0].reshape(NUM_TILES, N_BATCHES, EDGE_BATCH)
    vals = adj_vals.reshape(NUM_TILES, N_BATCHES, EDGE_BATCH)

    h1 = _mm1(x, W1)
    a1 = _segsum_l1(h1, col, row, vals)
    h2 = _mm2(a1, W2)
    return _segsum_l2(h2, col, row, vals)
